# Initial kernel scaffold; baseline (speedup 1.0000x reference)
import jax
import jax.numpy as jnp
from jax.experimental import pallas as pl


def kernel(x_main, x_guide, edge_index, u_basis, fg_w1, fg_b1, fg_w2, fg_b2, ino_s1_w, ino_s1_b, ino_t1_w, ino_t1_b, ino_s2_w, ino_s2_b, ino_t2_w, ino_t2_b, spa_w, spa_b, attn_w, attn_b, out_w, out_b, gamma):
    raise NotImplementedError("write your pallas kernel here")


# speedup vs baseline: 5.6883x; 5.6883x over previous
import functools

import jax
import jax.numpy as jnp
from jax import lax
from jax.experimental import pallas as pl
from jax.experimental.pallas import tpu as pltpu
from jax.experimental.pallas import tpu_sc as plsc

N = 10000
D = 128
E = 160000

NC = 2
NS = 16
CHUNK = 128
EPAD = 163840
CPT = EPAD // (NS * CHUNK)
NP = 10240
HALF = NP // NC
HRPT = HALF // NS
ACC_ROWS = HALF + 16


@functools.lru_cache(maxsize=None)
def _sc_kernels():
    mesh = plsc.VectorSubcoreMesh(core_axis_name="c", subcore_axis_name="s",
                                  num_cores=NC, num_subcores=NS)

    def _stage_local_dst(dst_v, dstc, j, c):
        base = c * HALF
        for q in range(CHUNK // 16):
            d = dst_v[j, pl.ds(q * 16, 16)] - base
            ok = (d >= 0) & (d < HALF)
            dstc[pl.ds(q * 16, 16)] = jnp.where(ok, d, HALF + q)

    @functools.partial(
        pl.kernel,
        out_type=jax.ShapeDtypeStruct((NP, D), jnp.float32),
        mesh=mesh,
        scratch_types=[
            pltpu.VMEM((CPT, CHUNK), jnp.int32),
            pltpu.VMEM((CHUNK,), jnp.int32),
            pltpu.VMEM((CHUNK, D), jnp.float32),
            pltpu.VMEM((HRPT, D), jnp.float32),
            pltpu.VMEM_SHARED((ACC_ROWS, D), jnp.float32),
        ],
    )
    def _deg_kernel(dsts_hbm, out_hbm, dst_v, dstc, ones_v, zero_v, acc):
        c = lax.axis_index("c")
        s = lax.axis_index("s")
        pltpu.sync_copy(dsts_hbm.at[s], dst_v)

        def fill_ones(i, _):
            for q in range(D // 16):
                ones_v[i, pl.ds(q * 16, 16)] = jnp.ones((16,), jnp.float32)
            return 0

        lax.fori_loop(0, CHUNK, fill_ones, 0)

        def fill_zero(i, _):
            for q in range(D // 16):
                zero_v[i, pl.ds(q * 16, 16)] = jnp.zeros((16,), jnp.float32)
            return 0

        lax.fori_loop(0, HRPT, fill_zero, 0)
        pltpu.sync_copy(zero_v, acc.at[pl.ds(s * HRPT, HRPT)])
        plsc.subcore_barrier()

        def body(j, _):
            _stage_local_dst(dst_v, dstc, j, c)
            pltpu.sync_copy(ones_v, acc.at[dstc], add=True)
            return 0

        lax.fori_loop(0, CPT, body, 0)
        plsc.subcore_barrier()
        pltpu.sync_copy(
            acc.at[pl.ds(s * HRPT, HRPT)],
            out_hbm.at[pl.ds(c * HALF + s * HRPT, HRPT)],
        )

    @functools.partial(
        pl.kernel,
        out_type=jax.ShapeDtypeStruct((NP, D), jnp.float32),
        mesh=mesh,
        scratch_types=[
            pltpu.VMEM((CPT, CHUNK), jnp.int32),
            pltpu.VMEM((CPT, CHUNK), jnp.int32),
            pltpu.VMEM((CHUNK,), jnp.int32),
            pltpu.VMEM((CHUNK,), jnp.int32),
            pltpu.VMEM((CHUNK,), jnp.int32),
            pltpu.VMEM((CHUNK, D), jnp.float32),
            pltpu.VMEM((CHUNK, D), jnp.float32),
            pltpu.VMEM_SHARED((ACC_ROWS, D), jnp.float32),
            pltpu.SemaphoreType.DMA,
            pltpu.SemaphoreType.DMA,
        ],
    )
    def _prop_kernel(xs_hbm, srcs_hbm, dsts_hbm, out_hbm,
                     src_v, dst_v, srca, srcb, dstc, bufa, bufb, acc,
                     sema, semb):
        c = lax.axis_index("c")
        s = lax.axis_index("s")
        pltpu.sync_copy(srcs_hbm.at[s], src_v)
        pltpu.sync_copy(dsts_hbm.at[s], dst_v)
        pltpu.sync_copy(
            xs_hbm.at[pl.ds(c * HALF + s * HRPT, HRPT)],
            acc.at[pl.ds(s * HRPT, HRPT)],
        )
        plsc.subcore_barrier()

        def body(i, _):
            j = 2 * i
            for q in range(CHUNK // 16):
                srca[pl.ds(q * 16, 16)] = src_v[j, pl.ds(q * 16, 16)]
                srcb[pl.ds(q * 16, 16)] = src_v[j + 1, pl.ds(q * 16, 16)]
            cpa = pltpu.async_copy(xs_hbm.at[srca], bufa, sema)
            cpb = pltpu.async_copy(xs_hbm.at[srcb], bufb, semb)
            _stage_local_dst(dst_v, dstc, j, c)
            cpa.wait()
            pltpu.sync_copy(bufa, acc.at[dstc], add=True)
            _stage_local_dst(dst_v, dstc, j + 1, c)
            cpb.wait()
            pltpu.sync_copy(bufb, acc.at[dstc], add=True)
            return 0

        lax.fori_loop(0, CPT // 2, body, 0)
        plsc.subcore_barrier()
        pltpu.sync_copy(
            acc.at[pl.ds(s * HRPT, HRPT)],
            out_hbm.at[pl.ds(c * HALF + s * HRPT, HRPT)],
        )

    return _deg_kernel, _prop_kernel


def _prep_body3(deg_ref, xg_ref, dinv_ref, xgt_ref):
    deg = deg_ref[:, :1] + 1.0
    dinv = lax.rsqrt(deg)
    dinv_ref[...] = dinv
    xgt_ref[...] = dinv * xg_ref[...]


def _gft1_body(u_ref, x_ref, o_ref):
    @pl.when(pl.program_id(1) == 0)
    def _():
        o_ref[...] = jnp.zeros_like(o_ref)

    o_ref[...] += lax.dot_general(
        u_ref[...], x_ref[...],
        dimension_numbers=(((0,), (0,)), ((), ())),
        preferred_element_type=jnp.float32,
    )


def _gft1_call(u, xc):
    bi, bk = 512, 5000
    grid = (pl.cdiv(N, bi), N // bk)
    return pl.pallas_call(
        _gft1_body,
        grid=grid,
        in_specs=[
            pl.BlockSpec((bk, bi), lambda i, k: (k, i)),
            pl.BlockSpec((bk, 2 * D), lambda i, k: (k, 0)),
        ],
        out_specs=pl.BlockSpec((bi, 2 * D), lambda i, k: (i, 0)),
        out_shape=jax.ShapeDtypeStruct((N, 2 * D), jnp.float32),
        compiler_params=pltpu.CompilerParams(
            dimension_semantics=("arbitrary", "arbitrary")),
    )(u, xc)


def _gft2_body(u_ref, z_ref, o_ref):
    o_ref[...] = jnp.dot(u_ref[...], z_ref[...],
                         preferred_element_type=jnp.float32)


def _gft2_call(u, z):
    bi = 400
    grid = (N // bi,)
    return pl.pallas_call(
        _gft2_body,
        grid=grid,
        in_specs=[
            pl.BlockSpec((bi, N), lambda i: (i, 0)),
            pl.BlockSpec((N, D), lambda i: (0, 0)),
        ],
        out_specs=pl.BlockSpec((bi, D), lambda i: (i, 0)),
        out_shape=jax.ShapeDtypeStruct((N, D), jnp.float32),
        compiler_params=pltpu.CompilerParams(
            dimension_semantics=("arbitrary",)),
    )(u, z)


def _mask_body(cat_ref, w1_ref, b1_ref, w2_ref, b2_ref, z_ref):
    cat = cat_ref[...]
    h1 = jnp.maximum(jnp.dot(cat, w1_ref[...],
                             preferred_element_type=jnp.float32) + b1_ref[...], 0.0)
    m = jax.nn.sigmoid(jnp.dot(h1, w2_ref[...],
                               preferred_element_type=jnp.float32) + b2_ref[...])
    z_ref[...] = cat[:, :D] + m * cat[:, D:]


def _mask_call(cat_hat, w1, b1, w2, b2):
    bm = 512
    grid = (pl.cdiv(N, bm),)
    return pl.pallas_call(
        _mask_body,
        grid=grid,
        in_specs=[
            pl.BlockSpec((bm, 2 * D), lambda i: (i, 0)),
            pl.BlockSpec((2 * D, D), lambda i: (0, 0)),
            pl.BlockSpec((1, D), lambda i: (0, 0)),
            pl.BlockSpec((D, D), lambda i: (0, 0)),
            pl.BlockSpec((1, D), lambda i: (0, 0)),
        ],
        out_specs=pl.BlockSpec((bm, D), lambda i: (i, 0)),
        out_shape=jax.ShapeDtypeStruct((N, D), jnp.float32),
    )(cat_hat, w1, b1, w2, b2)


def _round_body(a_ref, dinv_ref, xu_ref,
                ws_ref, bs_ref, wt_ref, bt_ref, xn_ref, xnt_ref):
    dinv = dinv_ref[...]
    p = dinv * a_ref[...]
    sgat = jnp.tanh(jnp.dot(p, ws_ref[...],
                            preferred_element_type=jnp.float32) + bs_ref[...])
    tgat = jnp.dot(p, wt_ref[...],
                   preferred_element_type=jnp.float32) + bt_ref[...]
    xn = xu_ref[...] * jnp.exp(sgat) + tgat
    xn_ref[...] = xn
    xnt_ref[...] = dinv * xn


def _round_call(acc, dinv2d, xupd, ws, bs, wt, bt):
    bm = 1024
    grid = (NP // bm,)
    return pl.pallas_call(
        _round_body,
        grid=grid,
        in_specs=[
            pl.BlockSpec((bm, D), lambda i: (i, 0)),
            pl.BlockSpec((bm, 1), lambda i: (i, 0)),
            pl.BlockSpec((bm, D), lambda i: (i, 0)),
            pl.BlockSpec((D, D), lambda i: (0, 0)),
            pl.BlockSpec((1, D), lambda i: (0, 0)),
            pl.BlockSpec((D, D), lambda i: (0, 0)),
            pl.BlockSpec((1, D), lambda i: (0, 0)),
        ],
        out_specs=[
            pl.BlockSpec((bm, D), lambda i: (i, 0)),
            pl.BlockSpec((bm, D), lambda i: (i, 0)),
        ],
        out_shape=[
            jax.ShapeDtypeStruct((NP, D), jnp.float32),
            jax.ShapeDtypeStruct((NP, D), jnp.float32),
        ],
    )(acc, dinv2d, xupd, ws, bs, wt, bt)


def _fuse_body(m_ref, g_ref, hf_ref, xm_ref, spaw_ref, spab_ref,
               attw_ref, attb_ref, outw_ref, outb_ref, gam_ref, o_ref):
    hcat = jnp.concatenate([m_ref[...], g_ref[...]], axis=1)
    h_spatial = jnp.dot(hcat, spaw_ref[...],
                        preferred_element_type=jnp.float32) + spab_ref[...]
    hf = hf_ref[...]
    f_detail = h_spatial - hf
    wi = jax.nn.sigmoid(
        jnp.dot(f_detail, attw_ref[...],
                preferred_element_type=jnp.float32) + attb_ref[...])
    h_enh = hf + gam_ref[...] * (wi * f_detail)
    ocat = jnp.concatenate([h_enh, h_spatial], axis=1)
    o_ref[...] = (jnp.dot(ocat, outw_ref[...],
                          preferred_element_type=jnp.float32)
                  + outb_ref[...] + xm_ref[...])


def _fuse_call(main, guide, h_freq, x_main, spa_w, spa_b,
               attn_w_col, attn_b, out_w, out_b, gamma11):
    bm = 1000
    grid = (N // bm,)
    return pl.pallas_call(
        _fuse_body,
        grid=grid,
        in_specs=[
            pl.BlockSpec((bm, D), lambda i: (i, 0)),
            pl.BlockSpec((bm, D), lambda i: (i, 0)),
            pl.BlockSpec((bm, D), lambda i: (i, 0)),
            pl.BlockSpec((bm, D), lambda i: (i, 0)),
            pl.BlockSpec((2 * D, D), lambda i: (0, 0)),
            pl.BlockSpec((1, D), lambda i: (0, 0)),
            pl.BlockSpec((D, 1), lambda i: (0, 0)),
            pl.BlockSpec((1, 1), lambda i: (0, 0)),
            pl.BlockSpec((2 * D, D), lambda i: (0, 0)),
            pl.BlockSpec((1, D), lambda i: (0, 0)),
            pl.BlockSpec((1, 1), lambda i: (0, 0)),
        ],
        out_specs=pl.BlockSpec((bm, D), lambda i: (i, 0)),
        out_shape=jax.ShapeDtypeStruct((N, D), jnp.float32),
    )(main, guide, h_freq, x_main, spa_w, spa_b,
      attn_w_col, attn_b, out_w, out_b, gamma11)


def kernel(x_main, x_guide, edge_index, u_basis, fg_w1, fg_b1, fg_w2, fg_b2,
           ino_s1_w, ino_s1_b, ino_t1_w, ino_t1_b, ino_s2_w, ino_s2_b,
           ino_t2_w, ino_t2_b, spa_w, spa_b, attn_w, attn_b, out_w, out_b,
           gamma):
    pad = EPAD - E
    src_p = jnp.concatenate(
        [edge_index[0], jnp.zeros((pad,), jnp.int32)]).reshape(NS, CPT, CHUNK)
    dst_p = jnp.concatenate(
        [edge_index[1], jnp.full((pad,), N, jnp.int32)]).reshape(NS, CPT, CHUNK)

    deg_k, prop_k = _sc_kernels()
    deg_full = deg_k(dst_p)

    bm = 1024
    dinv2d, xg_t = pl.pallas_call(
        _prep_body3,
        grid=(NP // bm,),
        in_specs=[
            pl.BlockSpec((bm, D), lambda i: (i, 0)),
            pl.BlockSpec((bm, D), lambda i: (i, 0)),
        ],
        out_specs=[
            pl.BlockSpec((bm, 1), lambda i: (i, 0)),
            pl.BlockSpec((bm, D), lambda i: (i, 0)),
        ],
        out_shape=[
            jax.ShapeDtypeStruct((NP, 1), jnp.float32),
            jax.ShapeDtypeStruct((NP, D), jnp.float32),
        ],
    )(deg_full, x_guide)

    cat = jnp.concatenate([x_main, x_guide], axis=1)
    cat_hat = _gft1_call(u_basis, cat)
    z = _mask_call(cat_hat, fg_w1, fg_b1.reshape(1, D), fg_w2,
                   fg_b2.reshape(1, D))
    h_freq = _gft2_call(u_basis, z)

    main, guide = x_main, x_guide
    xt = xg_t
    for i in range(3):
        acc = prop_k(xt, src_p, dst_p)
        main, xt = _round_call(acc, dinv2d, main,
                               ino_s1_w[i], ino_s1_b[i].reshape(1, D),
                               ino_t1_w[i], ino_t1_b[i].reshape(1, D))
        acc = prop_k(xt, src_p, dst_p)
        guide, xt = _round_call(acc, dinv2d, guide,
                                ino_s2_w[i], ino_s2_b[i].reshape(1, D),
                                ino_t2_w[i], ino_t2_b[i].reshape(1, D))

    return _fuse_call(main, guide, h_freq, x_main, spa_w, spa_b.reshape(1, D),
                      attn_w, attn_b.reshape(1, 1), out_w,
                      out_b.reshape(1, D), gamma.reshape(1, 1))
